# baseline (speedup 1.0000x reference)
import functools

import jax
import jax.numpy as jnp
from jax import lax
from jax.experimental import pallas as pl
from jax.experimental.pallas import tpu as pltpu
from jax.experimental.pallas import tpu_sc as plsc

NUM_ENT = 1000000
NUM_REL = 1000
EMB = 64
BATCH = 16384
NC = 2
NS = 16
NW = NC * NS
BPW = BATCH // NW
CH = 128
N_CH = BPW // CH
EPAIR = NUM_ENT // 2
RPAIR = NUM_REL // 2

_mesh = plsc.VectorSubcoreMesh(core_axis_name="c", subcore_axis_name="s")


@functools.partial(
    pl.kernel,
    out_type=jax.ShapeDtypeStruct((BATCH,), jnp.float32),
    mesh=_mesh,
    compiler_params=pltpu.CompilerParams(needs_layout_passes=False),
    scratch_types=[
        pltpu.VMEM((BPW,), jnp.int32),
        pltpu.VMEM((BPW,), jnp.int32),
        pltpu.VMEM((BPW,), jnp.int32),
        pltpu.VMEM((BPW,), jnp.int32),
        pltpu.VMEM((BPW,), jnp.int32),
        pltpu.VMEM((BPW,), jnp.int32),
        pltpu.VMEM((CH, 128), jnp.float32),
        pltpu.VMEM((CH, 128), jnp.float32),
        pltpu.VMEM((CH, 128), jnp.float32),
        pltpu.VMEM((CH, 128), jnp.float32),
        pltpu.VMEM((CH, 128), jnp.float32),
        pltpu.VMEM((CH, 128), jnp.float32),
        pltpu.VMEM((BPW,), jnp.float32),
        pltpu.SemaphoreType.DMA,
        pltpu.SemaphoreType.DMA,
        pltpu.SemaphoreType.DMA,
        pltpu.SemaphoreType.DMA,
        pltpu.SemaphoreType.DMA,
        pltpu.SemaphoreType.DMA,
    ],
)
def _bm_kernel(h_hbm, r_hbm, t_hbm, ent_hbm, rel_hbm, out_hbm,
               hp_v, rp_v, tp_v, hq_v, rq_v, tq_v,
               eh0_v, eh1_v, er0_v, er1_v, et0_v, et1_v,
               o_v, sh0, sh1, sr0, sr1, st0, st1):
    wid = lax.axis_index("s") * NC + lax.axis_index("c")
    base = wid * BPW

    pltpu.sync_copy(h_hbm.at[pl.ds(base, BPW)], hp_v)
    pltpu.sync_copy(r_hbm.at[pl.ds(base, BPW)], rp_v)
    pltpu.sync_copy(t_hbm.at[pl.ds(base, BPW)], tp_v)

    def split_body(j, carry):
        sl = pl.ds(j * 16, 16)
        for pv, qv in ((hp_v, hq_v), (rp_v, rq_v), (tp_v, tq_v)):
            e = pv[sl]
            pv[sl] = e >> 1
            qv[sl] = (e & 1) * EMB
        return carry

    lax.fori_loop(0, BPW // 16, split_body, 0, unroll=4)

    eh_b = (eh0_v, eh1_v)
    er_b = (er0_v, er1_v)
    et_b = (et0_v, et1_v)
    sh_b = (sh0, sh1)
    sr_b = (sr0, sr1)
    st_b = (st0, st1)

    def fire(c, p):
        isl = pl.ds(c * CH, CH)
        pltpu.async_copy(ent_hbm.at[hp_v.at[isl]], eh_b[p], sh_b[p])
        pltpu.async_copy(rel_hbm.at[rp_v.at[isl]], er_b[p], sr_b[p])
        pltpu.async_copy(ent_hbm.at[tp_v.at[isl]], et_b[p], st_b[p])

    def drain(p):
        pltpu.make_async_copy(
            ent_hbm.at[pl.ds(0, CH)], eh_b[p], sh_b[p]).wait()
        pltpu.make_async_copy(
            rel_hbm.at[pl.ds(0, CH)], er_b[p], sr_b[p]).wait()
        pltpu.make_async_copy(
            ent_hbm.at[pl.ds(0, CH)], et_b[p], st_b[p]).wait()

    lane_ids = lax.iota(jnp.int32, 16)

    def compute(c, p):
        ehv, erv, etv = eh_b[p], er_b[p], et_b[p]

        def grp_body(g, carry):
            sl = pl.ds(c * CH + g * 16, 16)
            rows = g * 16 + lane_ids
            hq = hq_v[sl]
            rq = rq_v[sl]
            tq = tq_v[sl]
            acc = jnp.zeros((16,), jnp.float32)
            for d in range(EMB):
                a = plsc.load_gather(ehv, [rows, hq + d])
                b = plsc.load_gather(erv, [rows, rq + d])
                cc = plsc.load_gather(etv, [rows, tq + d])
                acc = acc + a * b * cc
            o_v[sl] = 1.0 / (1.0 + jnp.exp(-acc))
            return carry

        lax.fori_loop(0, CH // 16, grp_body, 0)

    fire(0, 0)
    for c in range(N_CH):
        if c + 1 < N_CH:
            fire(c + 1, (c + 1) % 2)
        drain(c % 2)
        compute(c, c % 2)

    pltpu.sync_copy(o_v, out_hbm.at[pl.ds(base, BPW)])


def kernel(h, r, t, ent_table, rel_table):
    return _bm_kernel(
        h.astype(jnp.int32),
        r.astype(jnp.int32),
        t.astype(jnp.int32),
        ent_table.reshape(EPAIR, 128),
        rel_table.reshape(RPAIR, 128),
    )


# speedup vs baseline: 1.5884x; 1.5884x over previous
import functools

import jax
import jax.numpy as jnp
from jax import lax
from jax.experimental import pallas as pl
from jax.experimental.pallas import tpu as pltpu
from jax.experimental.pallas import tpu_sc as plsc

EMB = 64
BATCH = 16384
NC = 2
NS = 16
NW = NC * NS
BPW = BATCH // NW
CH = 128
N_CH = BPW // CH

_mesh = plsc.VectorSubcoreMesh(core_axis_name="c", subcore_axis_name="s")


@functools.partial(
    pl.kernel,
    out_type=jax.ShapeDtypeStruct((BATCH,), jnp.float32),
    mesh=_mesh,
    compiler_params=pltpu.CompilerParams(needs_layout_passes=False),
    scratch_types=[
        pltpu.VMEM((BPW,), jnp.int32),
        pltpu.VMEM((BPW,), jnp.int32),
        pltpu.VMEM((BPW,), jnp.int32),
        pltpu.VMEM((CH, EMB), jnp.float32),
        pltpu.VMEM((CH, EMB), jnp.float32),
        pltpu.VMEM((CH, EMB), jnp.float32),
        pltpu.VMEM((CH, EMB), jnp.float32),
        pltpu.VMEM((CH, EMB), jnp.float32),
        pltpu.VMEM((CH, EMB), jnp.float32),
        pltpu.VMEM((BPW,), jnp.float32),
        pltpu.SemaphoreType.DMA,
        pltpu.SemaphoreType.DMA,
        pltpu.SemaphoreType.DMA,
        pltpu.SemaphoreType.DMA,
        pltpu.SemaphoreType.DMA,
        pltpu.SemaphoreType.DMA,
    ],
)
def _bm_kernel(h_hbm, r_hbm, t_hbm, ent_hbm, rel_hbm, out_hbm,
               h_v, r_v, t_v,
               eh0_v, eh1_v, er0_v, er1_v, et0_v, et1_v,
               o_v, sh0, sh1, sr0, sr1, st0, st1):
    wid = lax.axis_index("s") * NC + lax.axis_index("c")
    base = wid * BPW

    pltpu.sync_copy(h_hbm.at[pl.ds(base, BPW)], h_v)
    pltpu.sync_copy(r_hbm.at[pl.ds(base, BPW)], r_v)
    pltpu.sync_copy(t_hbm.at[pl.ds(base, BPW)], t_v)

    eh_b = (eh0_v, eh1_v)
    er_b = (er0_v, er1_v)
    et_b = (et0_v, et1_v)
    sh_b = (sh0, sh1)
    sr_b = (sr0, sr1)
    st_b = (st0, st1)

    def fire(c, p):
        c0 = c * CH

        def fire_body(g, carry):
            hv = h_v[pl.ds(c0 + g * 16, 16)]
            rv = r_v[pl.ds(c0 + g * 16, 16)]
            tv = t_v[pl.ds(c0 + g * 16, 16)]
            for k in range(16):
                i = g * 16 + k
                pltpu.async_copy(ent_hbm.at[hv[k]], eh_b[p].at[i], sh_b[p])
                pltpu.async_copy(rel_hbm.at[rv[k]], er_b[p].at[i], sr_b[p])
                pltpu.async_copy(ent_hbm.at[tv[k]], et_b[p].at[i], st_b[p])
            return carry

        lax.fori_loop(0, CH // 16, fire_body, 0)

    def drain(p):
        pltpu.make_async_copy(
            ent_hbm.at[pl.ds(0, CH)], eh_b[p], sh_b[p]).wait()
        pltpu.make_async_copy(
            rel_hbm.at[pl.ds(0, CH)], er_b[p], sr_b[p]).wait()
        pltpu.make_async_copy(
            ent_hbm.at[pl.ds(0, CH)], et_b[p], st_b[p]).wait()

    lane_ids = lax.iota(jnp.int32, 16)

    def compute(c, p):
        ehv, erv, etv = eh_b[p], er_b[p], et_b[p]

        def grp_body(g, carry):
            rows = g * 16 + lane_ids
            acc = jnp.zeros((16,), jnp.float32)
            for d in range(EMB):
                col = jnp.full((16,), d, jnp.int32)
                a = plsc.load_gather(ehv, [rows, col])
                b = plsc.load_gather(erv, [rows, col])
                cc = plsc.load_gather(etv, [rows, col])
                acc = acc + a * b * cc
            o_v[pl.ds(c * CH + g * 16, 16)] = 1.0 / (1.0 + jnp.exp(-acc))
            return carry

        lax.fori_loop(0, CH // 16, grp_body, 0)

    fire(0, 0)
    for c in range(N_CH):
        if c + 1 < N_CH:
            fire(c + 1, (c + 1) % 2)
        drain(c % 2)
        compute(c, c % 2)

    pltpu.sync_copy(o_v, out_hbm.at[pl.ds(base, BPW)])


def kernel(h, r, t, ent_table, rel_table):
    return _bm_kernel(
        h.astype(jnp.int32),
        r.astype(jnp.int32),
        t.astype(jnp.int32),
        ent_table,
        rel_table,
    )
